# baseline (speedup 1.0000x reference)
import functools

import jax
import jax.numpy as jnp
from jax import lax
from jax.experimental import pallas as pl
from jax.experimental.pallas import tpu as pltpu
from jax.experimental.pallas import tpu_sc as plsc

_D_MODEL = 2048
_N_EXP = 16
_BLK = 1024
_TOKENS = 16384
_NEG = float(jnp.finfo(jnp.float32).min)

_NC = 2
_NS = 16
_NW = _NC * _NS
_TPW = _TOKENS // _NW
_L = 16
_GROUPS = _TPW // _L


def _matmul_body(x_ref, w_ref, l_ref):
    xb = x_ref[...]
    w = w_ref[...]
    l_ref[...] = jax.lax.dot_general(
        w, xb, (((1,), (1,)), ((), ())),
        preferred_element_type=jnp.float32)


def _tc_logits(x, W):
    tokens = x.shape[0]
    return pl.pallas_call(
        _matmul_body,
        grid=(tokens // _BLK,),
        in_specs=[
            pl.BlockSpec((_BLK, _D_MODEL), lambda i: (i, 0)),
            pl.BlockSpec((_N_EXP, _D_MODEL), lambda i: (0, 0)),
        ],
        out_specs=pl.BlockSpec((_N_EXP, _BLK), lambda i: (0, i)),
        out_shape=jax.ShapeDtypeStruct((_N_EXP, tokens), jnp.float32),
        compiler_params=pltpu.CompilerParams(
            dimension_semantics=("arbitrary",),
        ),
    )(x, W)


def _sc_body(l_hbm, g_hbm, i_hbm, lv, g1v, g2v, i1v, i2v, sem):
    wid = lax.axis_index("s") * _NC + lax.axis_index("c")
    base = wid * _TPW
    pltpu.sync_copy(l_hbm.at[:, pl.ds(base, _TPW)], lv)
    for g in range(_GROUPS):
        off = g * _L
        v = [lv[e, pl.ds(off, _L)] for e in range(_N_EXP)]
        m1 = v[0]
        for e in range(1, _N_EXP):
            m1 = jnp.maximum(m1, v[e])
        i1 = jnp.zeros((_L,), jnp.int32)
        for e in range(_N_EXP - 1, -1, -1):
            i1 = jnp.where(v[e] == m1, jnp.int32(e), i1)
        wv = [jnp.where(i1 == jnp.int32(e), _NEG, v[e]) for e in range(_N_EXP)]
        m2 = wv[0]
        for e in range(1, _N_EXP):
            m2 = jnp.maximum(m2, wv[e])
        i2 = jnp.zeros((_L,), jnp.int32)
        for e in range(_N_EXP - 1, -1, -1):
            i2 = jnp.where(wv[e] == m2, jnp.int32(e), i2)
        ex = jnp.exp(m2 - m1)
        g1v[pl.ds(off, _L)] = 1.0 / (1.0 + ex)
        g2v[pl.ds(off, _L)] = ex / (1.0 + ex)
        i1v[pl.ds(off, _L)] = i1
        i2v[pl.ds(off, _L)] = i2
    pltpu.sync_copy(g1v, g_hbm.at[0, pl.ds(base, _TPW)])
    pltpu.sync_copy(g2v, g_hbm.at[1, pl.ds(base, _TPW)])
    pltpu.sync_copy(i1v, i_hbm.at[0, pl.ds(base, _TPW)])
    pltpu.sync_copy(i2v, i_hbm.at[1, pl.ds(base, _TPW)])


def _sc_top2(logits_t):
    mesh = plsc.VectorSubcoreMesh(core_axis_name="c", subcore_axis_name="s")
    f = functools.partial(
        pl.kernel, _sc_body, mesh=mesh,
        out_type=[
            jax.ShapeDtypeStruct((2, _TOKENS), jnp.float32),
            jax.ShapeDtypeStruct((2, _TOKENS), jnp.int32),
        ],
        scratch_types=[
            pltpu.VMEM((_N_EXP, _TPW), jnp.float32),
            pltpu.VMEM((_TPW,), jnp.float32),
            pltpu.VMEM((_TPW,), jnp.float32),
            pltpu.VMEM((_TPW,), jnp.int32),
            pltpu.VMEM((_TPW,), jnp.int32),
            pltpu.SemaphoreType.DMA,
        ],
    )
    return f()(logits_t)


def kernel(x, W):
    logits_t = _tc_logits(x, W)
    gates_t, indices_t = _sc_top2(logits_t)
    return (gates_t.T, indices_t.T)
